# baseline (speedup 1.0000x reference)
import functools

import jax
import jax.numpy as jnp
from jax import lax
from jax.experimental import pallas as pl
from jax.experimental.pallas import tpu as pltpu
from jax.experimental.pallas import tpu_sc as plsc

B = 4096
F = 26
V = 100000
D = 16
C = 13

_NC = 2
_NS = 16
_NW = _NC * _NS
_BF = B * F
_BPW = _BF // _NW
_CH = 128
_NCH = _BPW // _CH


def _sc_gather(tables2d, idx2d):
    mesh = plsc.VectorSubcoreMesh(core_axis_name="c", subcore_axis_name="s")

    @functools.partial(
        pl.kernel,
        mesh=mesh,
        out_type=jax.ShapeDtypeStruct((_BF, D), jnp.float32),
        scratch_types=[
            pltpu.VMEM((_NCH, _CH), jnp.int32),
            pltpu.VMEM((_BPW, D), jnp.float32),
            pltpu.SemaphoreType.DMA,
        ],
        compiler_params=pltpu.CompilerParams(use_tc_tiling_on_sc=False),
    )
    def k(tbl_hbm, idx_hbm, out_hbm, idx_v, rows_v, sem):
        wid = lax.axis_index("s") * _NC + lax.axis_index("c")
        base = wid * _BPW
        pltpu.sync_copy(idx_hbm.at[wid], idx_v)

        def fire(j, carry):
            pltpu.async_copy(
                tbl_hbm.at[idx_v.at[j]],
                rows_v.at[pl.ds(j * _CH, _CH)],
                sem,
            )
            return carry

        lax.fori_loop(0, _NCH, fire, 0)
        pltpu.make_async_copy(tbl_hbm.at[pl.ds(0, _BPW)], rows_v, sem).wait()
        pltpu.sync_copy(rows_v, out_hbm.at[pl.ds(base, _BPW)])

    return k(tables2d, idx2d)


def _repack(tables_t):
    def body(in_ref, out_ref):
        x = in_ref[0]
        out_ref[0] = x.reshape(16, 128, 8).transpose(1, 2, 0).reshape(128, 128)

    return pl.pallas_call(
        body,
        grid=(F, 98),
        in_specs=[pl.BlockSpec((1, 16, 1024), lambda f, vb: (f, 0, vb))],
        out_specs=pl.BlockSpec((1, 128, 128), lambda f, vb: (f, vb, 0)),
        out_shape=jax.ShapeDtypeStruct((F, V // 8, 128), jnp.float32),
    )(tables_t)


def _mlp(emb, xc, w1e, w1c, b1, g1, be1, w2, b2, g2, be2, w3, b3, gc, bc):
    def body(emb_ref, xc_ref, w1e_ref, w1c_ref, b1_ref, g1_ref, be1_ref,
             w2_ref, b2_ref, g2_ref, be2_ref, w3_ref, b3_ref, gc_ref,
             bc_ref, out_ref):
        hp = jax.lax.Precision.HIGHEST
        x = xc_ref[...]
        m = jnp.mean(x, axis=0, keepdims=True)
        v = jnp.mean((x - m) * (x - m), axis=0, keepdims=True)
        xn = (x - m) * lax.rsqrt(v + 1e-5) * gc_ref[...] + bc_ref[...]

        h = jnp.dot(emb_ref[...], w1e_ref[...],
                    preferred_element_type=jnp.float32, precision=hp)
        h = h + jnp.dot(xn, w1c_ref[...],
                        preferred_element_type=jnp.float32, precision=hp)
        h = jnp.maximum(h + b1_ref[...], 0.0)
        m = jnp.mean(h, axis=0, keepdims=True)
        v = jnp.mean((h - m) * (h - m), axis=0, keepdims=True)
        h = (h - m) * lax.rsqrt(v + 1e-5) * g1_ref[...] + be1_ref[...]

        h = jnp.maximum(
            jnp.dot(h, w2_ref[...], preferred_element_type=jnp.float32,
                    precision=hp) + b2_ref[...], 0.0)
        m = jnp.mean(h, axis=0, keepdims=True)
        v = jnp.mean((h - m) * (h - m), axis=0, keepdims=True)
        h = (h - m) * lax.rsqrt(v + 1e-5) * g2_ref[...] + be2_ref[...]

        out_ref[...] = jnp.dot(
            h, w3_ref[...], preferred_element_type=jnp.float32,
            precision=hp) + b3_ref[...]

    return pl.pallas_call(
        body,
        out_shape=jax.ShapeDtypeStruct((B, 1), jnp.float32),
    )(emb, xc, w1e, w1c, b1, g1, be1, w2, b2, g2, be2, w3, b3, gc, bc)


def kernel(x_cont, x_emb, tables, W1, b1, g1, be1, W2, b2, g2, be2, W3, b3,
           gc, bc):
    packed = _repack(tables.transpose(0, 2, 1))
    tables2d = packed.reshape(F * V, D)
    offs = (jnp.arange(F, dtype=jnp.int32) * V)[None, :]
    idx3d = (x_emb + offs).reshape(_NW, _NCH, _CH)
    emb = _sc_gather(tables2d, idx3d).reshape(B, F * D)
    out = _mlp(
        emb, x_cont,
        W1[:F * D], W1[F * D:],
        b1.reshape(1, -1), g1.reshape(1, -1), be1.reshape(1, -1),
        W2, b2.reshape(1, -1), g2.reshape(1, -1), be2.reshape(1, -1),
        W3, b3.reshape(1, -1), gc.reshape(1, -1), bc.reshape(1, -1),
    )
    return out


# speedup vs baseline: 7.1856x; 7.1856x over previous
import functools

import jax
import jax.numpy as jnp
from jax import lax
from jax.experimental import pallas as pl
from jax.experimental.pallas import tpu as pltpu
from jax.experimental.pallas import tpu_sc as plsc

B = 4096
F = 26
V = 100000
D = 16
C = 13

_NW = 32
_R = F * D
_RPW = _R // _NW


def _sc_gather_t(tbl_lin, idx_t):
    mesh = plsc.VectorSubcoreMesh(core_axis_name="c", subcore_axis_name="s")

    @functools.partial(
        pl.kernel,
        mesh=mesh,
        out_type=jax.ShapeDtypeStruct((_R, B), jnp.float32),
        scratch_types=[
            pltpu.VMEM((V,), jnp.float32),
            pltpu.VMEM((B,), jnp.int32),
            pltpu.VMEM((B,), jnp.float32),
        ],
        compiler_params=pltpu.CompilerParams(
            use_tc_tiling_on_sc=False, needs_layout_passes=False),
    )
    def k(tbl_hbm, idx_hbm, out_hbm, row_v, idx_v, out_v):
        wid = lax.axis_index("s") * 2 + lax.axis_index("c")

        def do_row(i, carry):
            r = wid * _RPW + i
            f = r // D
            pltpu.sync_copy(tbl_hbm.at[r], row_v)
            pltpu.sync_copy(idx_hbm.at[f], idx_v)

            def extract(j, c2):
                vi = idx_v[pl.ds(j * 16, 16)]
                out_v[pl.ds(j * 16, 16)] = plsc.load_gather(row_v, [vi])
                return c2

            lax.fori_loop(0, B // 16, extract, 0)
            pltpu.sync_copy(out_v, out_hbm.at[r])
            return carry

        lax.fori_loop(0, _RPW, do_row, 0)

    return k(tbl_lin, idx_t)


def _mlp(embt, xc, w1e, w1c, b1, g1, be1, w2, b2, g2, be2, w3, b3, gc, bc):
    def body(embt_ref, xc_ref, w1e_ref, w1c_ref, b1_ref, g1_ref, be1_ref,
             w2_ref, b2_ref, g2_ref, be2_ref, w3_ref, b3_ref, gc_ref,
             bc_ref, out_ref):
        hp = jax.lax.Precision.HIGHEST
        x = xc_ref[...]
        m = jnp.mean(x, axis=0, keepdims=True)
        v = jnp.mean((x - m) * (x - m), axis=0, keepdims=True)
        xn = (x - m) * lax.rsqrt(v + 1e-5) * gc_ref[...] + bc_ref[...]

        h = jax.lax.dot_general(
            embt_ref[...], w1e_ref[...],
            dimension_numbers=(((0,), (0,)), ((), ())),
            preferred_element_type=jnp.float32, precision=hp)
        h = h + jnp.dot(xn, w1c_ref[...],
                        preferred_element_type=jnp.float32, precision=hp)
        h = jnp.maximum(h + b1_ref[...], 0.0)
        m = jnp.mean(h, axis=0, keepdims=True)
        v = jnp.mean((h - m) * (h - m), axis=0, keepdims=True)
        h = (h - m) * lax.rsqrt(v + 1e-5) * g1_ref[...] + be1_ref[...]

        h = jnp.maximum(
            jnp.dot(h, w2_ref[...], preferred_element_type=jnp.float32,
                    precision=hp) + b2_ref[...], 0.0)
        m = jnp.mean(h, axis=0, keepdims=True)
        v = jnp.mean((h - m) * (h - m), axis=0, keepdims=True)
        h = (h - m) * lax.rsqrt(v + 1e-5) * g2_ref[...] + be2_ref[...]

        out_ref[...] = jnp.dot(
            h, w3_ref[...], preferred_element_type=jnp.float32,
            precision=hp) + b3_ref[...]

    return pl.pallas_call(
        body,
        out_shape=jax.ShapeDtypeStruct((B, 1), jnp.float32),
    )(embt, xc, w1e, w1c, b1, g1, be1, w2, b2, g2, be2, w3, b3, gc, bc)


def kernel(x_cont, x_emb, tables, W1, b1, g1, be1, W2, b2, g2, be2, W3, b3,
           gc, bc):
    tbl_lin = tables.transpose(0, 2, 1).reshape(_R, V)
    embt = _sc_gather_t(tbl_lin, x_emb.T)
    out = _mlp(
        embt, x_cont,
        W1[:_R], W1[_R:],
        b1.reshape(1, -1), g1.reshape(1, -1), be1.reshape(1, -1),
        W2, b2.reshape(1, -1), g2.reshape(1, -1), be2.reshape(1, -1),
        W3, b3.reshape(1, -1), gc.reshape(1, -1), bc.reshape(1, -1),
    )
    return out
